# baseline (speedup 1.0000x reference)
import functools

import jax
import jax.numpy as jnp
from jax import lax
from jax.experimental import pallas as pl
from jax.experimental.pallas import tpu as pltpu
from jax.experimental.pallas import tpu_sc as plsc

_N = 16777216
_N_BINS = 10
_L = 16
_NC, _NS = 2, 16
_NW = _NC * _NS
_PER_W = _N // _NW
_CHUNK = 16384
_N_CHUNKS = _PER_W // _CHUNK
_VECS = _CHUNK // _L
_NSUB = 8

_mesh = plsc.VectorSubcoreMesh(
    core_axis_name="c", subcore_axis_name="s", num_cores=_NC, num_subcores=_NS
)


@functools.partial(
    pl.kernel,
    out_type=(
        jax.ShapeDtypeStruct((_NW, _N_BINS * _L), jnp.int32),
        jax.ShapeDtypeStruct((_NW, _N_BINS * _L), jnp.float32),
    ),
    mesh=_mesh,
    compiler_params=pltpu.CompilerParams(needs_layout_passes=False),
    scratch_types=[
        pltpu.VMEM((_CHUNK,), jnp.float32),
        pltpu.VMEM((_CHUNK,), jnp.int32),
        pltpu.VMEM((_CHUNK,), jnp.int32),
        pltpu.VMEM((_NSUB * _N_BINS * _L,), jnp.int32),
        pltpu.VMEM((_NSUB * _N_BINS * _L,), jnp.float32),
        pltpu.VMEM((_N_BINS * _L,), jnp.int32),
        pltpu.VMEM((_N_BINS * _L,), jnp.float32),
    ],
)
def _sc_hist(conf_hbm, pred_hbm, lab_hbm, pk_hbm, cf_hbm, conf_v, pred_v, lab_v, pk_v, cf_v,
             pko_v, cfo_v):
    wid = lax.axis_index("s") * _NC + lax.axis_index("c")
    for r in range(_NSUB * _N_BINS):
        pk_v[pl.ds(r * _L, _L)] = jnp.zeros((_L,), jnp.int32)
        cf_v[pl.ds(r * _L, _L)] = jnp.zeros((_L,), jnp.float32)
    lane = lax.iota(jnp.int32, _L)
    base0 = wid * _PER_W

    @pl.loop(0, _N_CHUNKS)
    def _chunks(ci):
        base = base0 + ci * _CHUNK
        pltpu.sync_copy(conf_hbm.at[pl.ds(base, _CHUNK)], conf_v)
        pltpu.sync_copy(pred_hbm.at[pl.ds(base, _CHUNK)], pred_v)
        pltpu.sync_copy(lab_hbm.at[pl.ds(base, _CHUNK)], lab_v)

        @pl.loop(0, _VECS, step=_NSUB)
        def _vecs(i0):
            for j in range(_NSUB):
                s = (i0 + j) * _L
                conf = conf_v[pl.ds(s, _L)]
                pred = pred_v[pl.ds(s, _L)]
                lab = lab_v[pl.ds(s, _L)]
                pk = jnp.where(pred == lab, 65537, 1).astype(jnp.int32)
                b = jnp.minimum((conf * 10.0).astype(jnp.int32), _N_BINS - 1)
                flat = b * _L + lane + (j * _N_BINS * _L)
                plsc.addupdate(pk_v.at[pl.ds(j * _N_BINS * _L, _L)], pk + flat)
                plsc.addupdate(cf_v.at[pl.ds(j * _N_BINS * _L, _L)], conf)

    for r in range(_N_BINS):
        dst = r * _L
        pk_t = pk_v[pl.ds(dst, _L)]
        cf_t = cf_v[pl.ds(dst, _L)]
        for j in range(1, _NSUB):
            src = j * _N_BINS * _L + r * _L
            pk_t = pk_t + pk_v[pl.ds(src, _L)]
            cf_t = cf_t + cf_v[pl.ds(src, _L)]
        pko_v[pl.ds(dst, _L)] = pk_t
        cfo_v[pl.ds(dst, _L)] = cf_t

    pltpu.sync_copy(pko_v, pk_hbm.at[wid])
    pltpu.sync_copy(cfo_v, cf_hbm.at[wid])


@functools.partial(
    pl.kernel,
    out_type=jax.ShapeDtypeStruct((_L,), jnp.float32),
    mesh=_mesh,
    compiler_params=pltpu.CompilerParams(needs_layout_passes=False),
    scratch_types=[
        pltpu.VMEM((_NW, _N_BINS * _L), jnp.int32),
        pltpu.VMEM((_NW, _N_BINS * _L), jnp.float32),
        pltpu.VMEM((_L,), jnp.float32),
    ],
)
def _sc_finalize(pk_hbm, cf_hbm, out_hbm, pkb_v, cfb_v, res_v):
    wid = lax.axis_index("s") * _NC + lax.axis_index("c")

    @pl.when(wid == 0)
    def _():
        pltpu.sync_copy(pk_hbm, pkb_v)
        pltpu.sync_copy(cf_hbm, cfb_v)
        lane = lax.iota(jnp.int32, _L)
        zeros = jnp.zeros((_L,), jnp.float32)
        cnt, accs, confs = zeros, zeros, zeros
        for b in range(_N_BINS):
            cnt_i = jnp.zeros((_L,), jnp.int32)
            acc_i = jnp.zeros((_L,), jnp.int32)
            cf_f = jnp.zeros((_L,), jnp.float32)
            for w in range(_NW):
                pk16 = pkb_v[w, pl.ds(b * _L, _L)]
                cnt_i = cnt_i + (pk16 & 0xFFFF)
                acc_i = acc_i + lax.shift_right_logical(pk16, 16)
                cf_f = cf_f + cfb_v[w, pl.ds(b * _L, _L)]
            cnt = cnt + jnp.where(lane == b, jnp.sum(cnt_i).astype(jnp.float32), 0.0)
            accs = accs + jnp.where(lane == b, jnp.sum(acc_i).astype(jnp.float32), 0.0)
            confs = confs + jnp.where(lane == b, jnp.sum(cf_f), 0.0)
        safe = jnp.maximum(cnt, 1.0)
        avg_acc = accs / safe
        avg_conf = confs / safe
        contrib = jnp.abs(avg_conf - avg_acc) * (cnt * (1.0 / _N))
        contrib = jnp.where(cnt > 0.0, contrib, 0.0)
        ece = jnp.sum(contrib)
        res_v[:] = zeros + ece
        pltpu.sync_copy(res_v, out_hbm)


def kernel(confidences, predictions, labels):
    pred = predictions.astype(jnp.int32)
    lab = labels.astype(jnp.int32)
    pk, cf = _sc_hist(confidences, pred, lab)
    out = _sc_finalize(pk, cf)
    return out[0]


# speedup vs baseline: 1.9216x; 1.9216x over previous
import functools

import jax
import jax.numpy as jnp
from jax import lax
from jax.experimental import pallas as pl
from jax.experimental.pallas import tpu as pltpu
from jax.experimental.pallas import tpu_sc as plsc

_N = 16777216
_N_BINS = 10
_L = 16
_NC, _NS = 2, 16
_NW = _NC * _NS
_PER_W = _N // _NW
_CHUNK = 16384
_N_CHUNKS = _PER_W // _CHUNK
_VECS = _CHUNK // _L
_NSUB = 8

_mesh = plsc.VectorSubcoreMesh(
    core_axis_name="c", subcore_axis_name="s", num_cores=_NC, num_subcores=_NS
)


@functools.partial(
    pl.kernel,
    out_type=(
        jax.ShapeDtypeStruct((_NW, _N_BINS * _L), jnp.int32),
        jax.ShapeDtypeStruct((_NW, _N_BINS * _L), jnp.float32),
    ),
    mesh=_mesh,
    compiler_params=pltpu.CompilerParams(needs_layout_passes=False),
    scratch_types=[
        pltpu.VMEM((_CHUNK,), jnp.float32),
        pltpu.VMEM((_CHUNK,), jnp.int32),
        pltpu.VMEM((_CHUNK,), jnp.int32),
        pltpu.VMEM((_NSUB * _N_BINS * _L,), jnp.int32),
        pltpu.VMEM((_NSUB * _N_BINS * _L,), jnp.float32),
        pltpu.VMEM((_N_BINS * _L,), jnp.int32),
        pltpu.VMEM((_N_BINS * _L,), jnp.float32),
    ],
)
def _sc_hist(conf_hbm, pred_hbm, lab_hbm, pk_hbm, cf_hbm, conf_v, pred_v, lab_v, pk_v, cf_v,
             pko_v, cfo_v):
    wid = lax.axis_index("s") * _NC + lax.axis_index("c")
    for r in range(_NSUB * _N_BINS):
        pk_v[pl.ds(r * _L, _L)] = jnp.zeros((_L,), jnp.int32)
        cf_v[pl.ds(r * _L, _L)] = jnp.zeros((_L,), jnp.float32)
    lane = lax.iota(jnp.int32, _L)
    base0 = wid * _PER_W

    @pl.loop(0, _N_CHUNKS)
    def _chunks(ci):
        base = base0 + ci * _CHUNK
        pltpu.sync_copy(conf_hbm.at[pl.ds(base, _CHUNK)], conf_v)
        pltpu.sync_copy(pred_hbm.at[pl.ds(base, _CHUNK)], pred_v)
        pltpu.sync_copy(lab_hbm.at[pl.ds(base, _CHUNK)], lab_v)

        @plsc.parallel_loop(0, _VECS, _NSUB)
        def _vecs(i0):
            for j in range(_NSUB):
                s = (i0 + j) * _L
                conf = conf_v[pl.ds(s, _L)]
                pred = pred_v[pl.ds(s, _L)]
                lab = lab_v[pl.ds(s, _L)]
                pk = jnp.where(pred == lab, 65537, 1).astype(jnp.int32)
                b = jnp.minimum((conf * 10.0).astype(jnp.int32), _N_BINS - 1)
                flat = b * _L + lane + (j * _N_BINS * _L)
                plsc.addupdate_scatter(pk_v, [flat], pk)
                plsc.addupdate_scatter(cf_v, [flat], conf)

    for r in range(_N_BINS):
        dst = r * _L
        pk_t = pk_v[pl.ds(dst, _L)]
        cf_t = cf_v[pl.ds(dst, _L)]
        for j in range(1, _NSUB):
            src = j * _N_BINS * _L + r * _L
            pk_t = pk_t + pk_v[pl.ds(src, _L)]
            cf_t = cf_t + cf_v[pl.ds(src, _L)]
        pko_v[pl.ds(dst, _L)] = pk_t
        cfo_v[pl.ds(dst, _L)] = cf_t

    pltpu.sync_copy(pko_v, pk_hbm.at[wid])
    pltpu.sync_copy(cfo_v, cf_hbm.at[wid])


@functools.partial(
    pl.kernel,
    out_type=jax.ShapeDtypeStruct((_L,), jnp.float32),
    mesh=_mesh,
    compiler_params=pltpu.CompilerParams(needs_layout_passes=False),
    scratch_types=[
        pltpu.VMEM((_NW, _N_BINS * _L), jnp.int32),
        pltpu.VMEM((_NW, _N_BINS * _L), jnp.float32),
        pltpu.VMEM((_L,), jnp.float32),
    ],
)
def _sc_finalize(pk_hbm, cf_hbm, out_hbm, pkb_v, cfb_v, res_v):
    wid = lax.axis_index("s") * _NC + lax.axis_index("c")

    @pl.when(wid == 0)
    def _():
        pltpu.sync_copy(pk_hbm, pkb_v)
        pltpu.sync_copy(cf_hbm, cfb_v)
        lane = lax.iota(jnp.int32, _L)
        zeros = jnp.zeros((_L,), jnp.float32)
        cnt, accs, confs = zeros, zeros, zeros
        for b in range(_N_BINS):
            cnt_i = jnp.zeros((_L,), jnp.int32)
            acc_i = jnp.zeros((_L,), jnp.int32)
            cf_f = jnp.zeros((_L,), jnp.float32)
            for w in range(_NW):
                pk16 = pkb_v[w, pl.ds(b * _L, _L)]
                cnt_i = cnt_i + (pk16 & 0xFFFF)
                acc_i = acc_i + lax.shift_right_logical(pk16, 16)
                cf_f = cf_f + cfb_v[w, pl.ds(b * _L, _L)]
            cnt = cnt + jnp.where(lane == b, jnp.sum(cnt_i).astype(jnp.float32), 0.0)
            accs = accs + jnp.where(lane == b, jnp.sum(acc_i).astype(jnp.float32), 0.0)
            confs = confs + jnp.where(lane == b, jnp.sum(cf_f), 0.0)
        safe = jnp.maximum(cnt, 1.0)
        avg_acc = accs / safe
        avg_conf = confs / safe
        contrib = jnp.abs(avg_conf - avg_acc) * (cnt * (1.0 / _N))
        contrib = jnp.where(cnt > 0.0, contrib, 0.0)
        ece = jnp.sum(contrib)
        res_v[:] = zeros + ece
        pltpu.sync_copy(res_v, out_hbm)


def kernel(confidences, predictions, labels):
    pred = predictions.astype(jnp.int32)
    lab = labels.astype(jnp.int32)
    pk, cf = _sc_hist(confidences, pred, lab)
    out = _sc_finalize(pk, cf)
    return out[0]


# speedup vs baseline: 2.4695x; 1.2851x over previous
import functools

import jax
import jax.numpy as jnp
from jax import lax
from jax.experimental import pallas as pl
from jax.experimental.pallas import tpu as pltpu
from jax.experimental.pallas import tpu_sc as plsc

_N = 16777216
_N_BINS = 10
_L = 16
_NC, _NS = 2, 16
_NW = _NC * _NS
_PER_W = _N // _NW
_CHUNK = 16384
_N_CHUNKS = _PER_W // _CHUNK
_VECS = _CHUNK // _L
_NSUB = 8

_mesh = plsc.VectorSubcoreMesh(
    core_axis_name="c", subcore_axis_name="s", num_cores=_NC, num_subcores=_NS
)


@functools.partial(
    pl.kernel,
    out_type=(
        jax.ShapeDtypeStruct((_NW, _N_BINS * _L), jnp.int32),
        jax.ShapeDtypeStruct((_NW, _N_BINS * _L), jnp.float32),
    ),
    mesh=_mesh,
    compiler_params=pltpu.CompilerParams(needs_layout_passes=False),
    scratch_types=[
        pltpu.VMEM((2, _CHUNK), jnp.float32),
        pltpu.VMEM((2, _CHUNK), jnp.int32),
        pltpu.VMEM((2, _CHUNK), jnp.int32),
        pltpu.VMEM((_NSUB * _N_BINS * _L,), jnp.int32),
        pltpu.VMEM((_NSUB * _N_BINS * _L,), jnp.float32),
        pltpu.VMEM((_N_BINS * _L,), jnp.int32),
        pltpu.VMEM((_N_BINS * _L,), jnp.float32),
        pltpu.SemaphoreType.DMA((2,)),
    ],
)
def _sc_hist(conf_hbm, pred_hbm, lab_hbm, pk_hbm, cf_hbm, conf_v, pred_v, lab_v, pk_v, cf_v,
             pko_v, cfo_v, sem):
    wid = lax.axis_index("s") * _NC + lax.axis_index("c")
    for r in range(_NSUB * _N_BINS):
        pk_v[pl.ds(r * _L, _L)] = jnp.zeros((_L,), jnp.int32)
        cf_v[pl.ds(r * _L, _L)] = jnp.zeros((_L,), jnp.float32)
    lane = lax.iota(jnp.int32, _L)
    base0 = wid * _PER_W

    def _start(ci, b):
        base = base0 + ci * _CHUNK
        pltpu.async_copy(conf_hbm.at[pl.ds(base, _CHUNK)], conf_v.at[b], sem.at[b])
        pltpu.async_copy(pred_hbm.at[pl.ds(base, _CHUNK)], pred_v.at[b], sem.at[b])
        pltpu.async_copy(lab_hbm.at[pl.ds(base, _CHUNK)], lab_v.at[b], sem.at[b])

    def _wait(ci, b):
        base = base0 + ci * _CHUNK
        pltpu.make_async_copy(conf_hbm.at[pl.ds(base, _CHUNK)], conf_v.at[b], sem.at[b]).wait()
        pltpu.make_async_copy(pred_hbm.at[pl.ds(base, _CHUNK)], pred_v.at[b], sem.at[b]).wait()
        pltpu.make_async_copy(lab_hbm.at[pl.ds(base, _CHUNK)], lab_v.at[b], sem.at[b]).wait()

    _start(0, 0)

    @pl.loop(0, _N_CHUNKS, step=2)
    def _chunks(ci):
        for b in range(2):
            cur = ci + b

            @pl.when(cur + 1 < _N_CHUNKS)
            def _():
                _start(cur + 1, 1 - b)

            _wait(cur, b)

            @plsc.parallel_loop(0, _VECS, _NSUB)
            def _vecs(i0):
                for j in range(_NSUB):
                    s = (i0 + j) * _L
                    conf = conf_v[b, pl.ds(s, _L)]
                    pred = pred_v[b, pl.ds(s, _L)]
                    lab = lab_v[b, pl.ds(s, _L)]
                    pk = jnp.where(pred == lab, 65537, 1).astype(jnp.int32)
                    bn = jnp.minimum((conf * 10.0).astype(jnp.int32), _N_BINS - 1)
                    flat = bn * _L + lane + (j * _N_BINS * _L)
                    plsc.addupdate_scatter(pk_v, [flat], pk)
                    plsc.addupdate_scatter(cf_v, [flat], conf)

    for r in range(_N_BINS):
        dst = r * _L
        pk_t = pk_v[pl.ds(dst, _L)]
        cf_t = cf_v[pl.ds(dst, _L)]
        for j in range(1, _NSUB):
            src = j * _N_BINS * _L + r * _L
            pk_t = pk_t + pk_v[pl.ds(src, _L)]
            cf_t = cf_t + cf_v[pl.ds(src, _L)]
        pko_v[pl.ds(dst, _L)] = pk_t
        cfo_v[pl.ds(dst, _L)] = cf_t

    pltpu.sync_copy(pko_v, pk_hbm.at[wid])
    pltpu.sync_copy(cfo_v, cf_hbm.at[wid])


@functools.partial(
    pl.kernel,
    out_type=jax.ShapeDtypeStruct((_L,), jnp.float32),
    mesh=_mesh,
    compiler_params=pltpu.CompilerParams(needs_layout_passes=False),
    scratch_types=[
        pltpu.VMEM((_NW, _N_BINS * _L), jnp.int32),
        pltpu.VMEM((_NW, _N_BINS * _L), jnp.float32),
        pltpu.VMEM((_L,), jnp.float32),
    ],
)
def _sc_finalize(pk_hbm, cf_hbm, out_hbm, pkb_v, cfb_v, res_v):
    wid = lax.axis_index("s") * _NC + lax.axis_index("c")

    @pl.when(wid == 0)
    def _():
        pltpu.sync_copy(pk_hbm, pkb_v)
        pltpu.sync_copy(cf_hbm, cfb_v)
        lane = lax.iota(jnp.int32, _L)
        zeros = jnp.zeros((_L,), jnp.float32)
        cnt, accs, confs = zeros, zeros, zeros
        for b in range(_N_BINS):
            cnt_i = jnp.zeros((_L,), jnp.int32)
            acc_i = jnp.zeros((_L,), jnp.int32)
            cf_f = jnp.zeros((_L,), jnp.float32)
            for w in range(_NW):
                pk16 = pkb_v[w, pl.ds(b * _L, _L)]
                cnt_i = cnt_i + (pk16 & 0xFFFF)
                acc_i = acc_i + lax.shift_right_logical(pk16, 16)
                cf_f = cf_f + cfb_v[w, pl.ds(b * _L, _L)]
            cnt = cnt + jnp.where(lane == b, jnp.sum(cnt_i).astype(jnp.float32), 0.0)
            accs = accs + jnp.where(lane == b, jnp.sum(acc_i).astype(jnp.float32), 0.0)
            confs = confs + jnp.where(lane == b, jnp.sum(cf_f), 0.0)
        safe = jnp.maximum(cnt, 1.0)
        avg_acc = accs / safe
        avg_conf = confs / safe
        contrib = jnp.abs(avg_conf - avg_acc) * (cnt * (1.0 / _N))
        contrib = jnp.where(cnt > 0.0, contrib, 0.0)
        ece = jnp.sum(contrib)
        res_v[:] = zeros + ece
        pltpu.sync_copy(res_v, out_hbm)


def kernel(confidences, predictions, labels):
    pred = predictions.astype(jnp.int32)
    lab = labels.astype(jnp.int32)
    pk, cf = _sc_hist(confidences, pred, lab)
    out = _sc_finalize(pk, cf)
    return out[0]


# speedup vs baseline: 5.5158x; 2.2335x over previous
import functools

import jax
import jax.numpy as jnp
from jax import lax
from jax.experimental import pallas as pl
from jax.experimental.pallas import tpu as pltpu
from jax.experimental.pallas import tpu_sc as plsc

_N = 16777216
_N_BINS = 10
_L = 16
_NC, _NS = 2, 16
_NW = _NC * _NS
_PER_W = _N // _NW
_CHUNK = 16384
_N_CHUNKS = _PER_W // _CHUNK
_VECS = _CHUNK // _L
_NSUB = 8

_mesh = plsc.VectorSubcoreMesh(
    core_axis_name="c", subcore_axis_name="s", num_cores=_NC, num_subcores=_NS
)


@functools.partial(
    pl.kernel,
    out_type=(
        jax.ShapeDtypeStruct((_NW, _N_BINS * _L), jnp.int32),
        jax.ShapeDtypeStruct((_NW, _N_BINS * _L), jnp.float32),
    ),
    mesh=_mesh,
    compiler_params=pltpu.CompilerParams(needs_layout_passes=False),
    scratch_types=[
        pltpu.VMEM((2, _CHUNK), jnp.float32),
        pltpu.VMEM((2, _CHUNK), jnp.int32),
        pltpu.VMEM((2, _CHUNK), jnp.int32),
        pltpu.VMEM((_NSUB * _N_BINS * _L,), jnp.int32),
        pltpu.VMEM((_NSUB * _N_BINS * _L,), jnp.float32),
        pltpu.VMEM((_N_BINS * _L,), jnp.int32),
        pltpu.VMEM((_N_BINS * _L,), jnp.float32),
        pltpu.SemaphoreType.DMA((2,)),
    ],
)
def _sc_hist(conf_hbm, pred_hbm, lab_hbm, pk_hbm, cf_hbm, conf_v, pred_v, lab_v, pk_v, cf_v,
             pko_v, cfo_v, sem):
    wid = lax.axis_index("s") * _NC + lax.axis_index("c")
    for r in range(_NSUB * _N_BINS):
        pk_v[pl.ds(r * _L, _L)] = jnp.zeros((_L,), jnp.int32)
        cf_v[pl.ds(r * _L, _L)] = jnp.zeros((_L,), jnp.float32)
    lane = lax.iota(jnp.int32, _L)
    base0 = wid * _PER_W

    def _start(ci, b):
        base = base0 + ci * _CHUNK
        pltpu.async_copy(conf_hbm.at[pl.ds(base, _CHUNK)], conf_v.at[b], sem.at[b])
        pltpu.async_copy(pred_hbm.at[pl.ds(base, _CHUNK)], pred_v.at[b], sem.at[b])
        pltpu.async_copy(lab_hbm.at[pl.ds(base, _CHUNK)], lab_v.at[b], sem.at[b])

    def _wait(ci, b):
        base = base0 + ci * _CHUNK
        pltpu.make_async_copy(conf_hbm.at[pl.ds(base, _CHUNK)], conf_v.at[b], sem.at[b]).wait()
        pltpu.make_async_copy(pred_hbm.at[pl.ds(base, _CHUNK)], pred_v.at[b], sem.at[b]).wait()
        pltpu.make_async_copy(lab_hbm.at[pl.ds(base, _CHUNK)], lab_v.at[b], sem.at[b]).wait()

    _start(0, 0)

    @pl.loop(0, _N_CHUNKS, step=2)
    def _chunks(ci):
        for b in range(2):
            cur = ci + b

            @pl.when(cur + 1 < _N_CHUNKS)
            def _():
                _start(cur + 1, 1 - b)

            _wait(cur, b)

            @functools.partial(plsc.parallel_loop, 0, _VECS, unroll=_NSUB)
            def _vecs(i):
                s = i * _L
                conf = conf_v[b, pl.ds(s, _L)]
                pred = pred_v[b, pl.ds(s, _L)]
                lab = lab_v[b, pl.ds(s, _L)]
                pk = jnp.where(pred == lab, 65537, 1).astype(jnp.int32)
                bn = jnp.minimum((conf * 10.0).astype(jnp.int32), _N_BINS - 1)
                flat = bn * _L + lane + (i & (_NSUB - 1)) * (_N_BINS * _L)
                plsc.addupdate_scatter(pk_v, [flat], pk)
                plsc.addupdate_scatter(cf_v, [flat], conf)

    for r in range(_N_BINS):
        dst = r * _L
        pk_t = pk_v[pl.ds(dst, _L)]
        cf_t = cf_v[pl.ds(dst, _L)]
        for j in range(1, _NSUB):
            src = j * _N_BINS * _L + r * _L
            pk_t = pk_t + pk_v[pl.ds(src, _L)]
            cf_t = cf_t + cf_v[pl.ds(src, _L)]
        pko_v[pl.ds(dst, _L)] = pk_t
        cfo_v[pl.ds(dst, _L)] = cf_t

    pltpu.sync_copy(pko_v, pk_hbm.at[wid])
    pltpu.sync_copy(cfo_v, cf_hbm.at[wid])


@functools.partial(
    pl.kernel,
    out_type=jax.ShapeDtypeStruct((_L,), jnp.float32),
    mesh=_mesh,
    compiler_params=pltpu.CompilerParams(needs_layout_passes=False),
    scratch_types=[
        pltpu.VMEM((_NW, _N_BINS * _L), jnp.int32),
        pltpu.VMEM((_NW, _N_BINS * _L), jnp.float32),
        pltpu.VMEM((_L,), jnp.float32),
    ],
)
def _sc_finalize(pk_hbm, cf_hbm, out_hbm, pkb_v, cfb_v, res_v):
    wid = lax.axis_index("s") * _NC + lax.axis_index("c")

    @pl.when(wid == 0)
    def _():
        pltpu.sync_copy(pk_hbm, pkb_v)
        pltpu.sync_copy(cf_hbm, cfb_v)
        lane = lax.iota(jnp.int32, _L)
        zeros = jnp.zeros((_L,), jnp.float32)
        cnt, accs, confs = zeros, zeros, zeros
        for b in range(_N_BINS):
            cnt_i = jnp.zeros((_L,), jnp.int32)
            acc_i = jnp.zeros((_L,), jnp.int32)
            cf_f = jnp.zeros((_L,), jnp.float32)
            for w in range(_NW):
                pk16 = pkb_v[w, pl.ds(b * _L, _L)]
                cnt_i = cnt_i + (pk16 & 0xFFFF)
                acc_i = acc_i + lax.shift_right_logical(pk16, 16)
                cf_f = cf_f + cfb_v[w, pl.ds(b * _L, _L)]
            cnt = cnt + jnp.where(lane == b, jnp.sum(cnt_i).astype(jnp.float32), 0.0)
            accs = accs + jnp.where(lane == b, jnp.sum(acc_i).astype(jnp.float32), 0.0)
            confs = confs + jnp.where(lane == b, jnp.sum(cf_f), 0.0)
        safe = jnp.maximum(cnt, 1.0)
        avg_acc = accs / safe
        avg_conf = confs / safe
        contrib = jnp.abs(avg_conf - avg_acc) * (cnt * (1.0 / _N))
        contrib = jnp.where(cnt > 0.0, contrib, 0.0)
        ece = jnp.sum(contrib)
        res_v[:] = zeros + ece
        pltpu.sync_copy(res_v, out_hbm)


def kernel(confidences, predictions, labels):
    pred = predictions.astype(jnp.int32)
    lab = labels.astype(jnp.int32)
    pk, cf = _sc_hist(confidences, pred, lab)
    out = _sc_finalize(pk, cf)
    return out[0]
